# Initial kernel scaffold; baseline (speedup 1.0000x reference)
import jax
import jax.numpy as jnp
from jax.experimental import pallas as pl


def kernel(x, W):
    raise NotImplementedError("write your pallas kernel here")


# speedup vs baseline: 25.2762x; 25.2762x over previous
import jax
import jax.numpy as jnp
from jax.experimental import pallas as pl
from jax.experimental.pallas import tpu as pltpu

K = 64
RB = 128
N_ITERS = 26


def _body(x_ref, wt_ref, z_ref, s_ref):
    x = x_ref[...]
    norm2 = jnp.sum(x * x, axis=1, keepdims=True)
    xn = x * jax.lax.rsqrt(jnp.maximum(norm2, 1e-24))
    s = jnp.dot(xn, wt_ref[...], preferred_element_type=jnp.float32)
    s_ref[...] = s

    hi = jnp.max(s, axis=1, keepdims=True) + 1e-6
    lo = jnp.min(s, axis=1, keepdims=True) - 1e-6

    def step(_, carry):
        lo, hi = carry
        mid = (lo + hi) * 0.5
        cnt = jnp.sum((s_ref[...] >= mid).astype(jnp.float32), axis=1,
                      keepdims=True)
        ge = cnt >= K
        return jnp.where(ge, mid, lo), jnp.where(ge, hi, mid)

    lo, hi = jax.lax.fori_loop(0, N_ITERS, step, (lo, hi))
    s = s_ref[...]
    z_ref[...] = jnp.where(s >= lo, jnp.maximum(s, 0.0), 0.0)


def kernel(x, W):
    B, D = x.shape
    N = W.shape[0]
    wt = W.T
    return pl.pallas_call(
        _body,
        grid=(B // RB,),
        in_specs=[
            pl.BlockSpec((RB, D), lambda i: (i, 0)),
            pl.BlockSpec((D, N), lambda i: (0, 0)),
        ],
        out_specs=pl.BlockSpec((RB, N), lambda i: (i, 0)),
        out_shape=jax.ShapeDtypeStruct((B, N), jnp.float32),
        scratch_shapes=[pltpu.VMEM((RB, N), jnp.float32)],
    )(x, wt)
